# baseline (speedup 1.0000x reference)
import functools

import jax
import jax.numpy as jnp
from jax import lax
from jax.experimental import pallas as pl
from jax.experimental.pallas import tpu as pltpu
from jax.experimental.pallas import tpu_sc as plsc

N_NODES = 10000
N_EDGES = 320000
IN_CH = 128
OUT_CH = 64

NC = 2
NS = 16
NW = NC * NS
NP = 10240
EPW = N_EDGES // NW
CH = 80
NCHUNK = EPW // CH
SLICE = NP // NS
CHA = 400
CW = 128

_mesh = plsc.VectorSubcoreMesh(
    core_axis_name="c", subcore_axis_name="s", num_cores=NC, num_subcores=NS)


@functools.partial(
    pl.kernel,
    out_type=jax.ShapeDtypeStruct((NC, NP), jnp.float32),
    mesh=_mesh,
    scratch_types=[
        pltpu.VMEM((NCHUNK, CH), jnp.int32),
        pltpu.VMEM((NP,), jnp.float32),
        pltpu.VMEM_SHARED((NS, NP), jnp.float32),
        pltpu.VMEM((SLICE,), jnp.float32),
        pltpu.VMEM((SLICE,), jnp.float32),
        pltpu.VMEM((SLICE,), jnp.float32),
        pltpu.SemaphoreType.DMA,
        pltpu.SemaphoreType.DMA,
        pltpu.SemaphoreType.DMA,
    ],
    compiler_params=pltpu.CompilerParams(
        needs_layout_passes=False, use_tc_tiling_on_sc=False),
)
def _degree_kernel(e3_hbm, out_hbm, dstall, hist, shared, acc, tmp0, tmp1,
                   si, st0, st1):
    c = lax.axis_index("c")
    s = lax.axis_index("s")
    wid = s * NC + c
    zeros16 = jnp.zeros((16,), jnp.float32)
    ones16 = jnp.ones((16,), jnp.float32)

    pltpu.async_copy(e3_hbm.at[1, wid], dstall, si)

    def zero_hist(i, _):
        hist[pl.ds(i * 16, 16)] = zeros16
        return ()
    lax.fori_loop(0, NP // 16, zero_hist, ())
    pltpu.make_async_copy(e3_hbm.at[1, wid], dstall, si).wait()

    def outer(j, _):
        def inner(k, _):
            idx = dstall[j, pl.ds(k * 16, 16)]
            plsc.addupdate_scatter(hist, [idx], ones16)
            return ()
        lax.fori_loop(0, CH // 16, inner, ())
        return ()
    lax.fori_loop(0, NCHUNK, outer, ())

    pltpu.sync_copy(hist, shared.at[s])
    plsc.subcore_barrier()

    def zero_acc(i, _):
        acc[pl.ds(i * 16, 16)] = zeros16
        return ()
    lax.fori_loop(0, SLICE // 16, zero_acc, ())

    sl = pl.ds(s * SLICE, SLICE)
    pltpu.async_copy(shared.at[0, sl], tmp0, st0)
    pltpu.async_copy(shared.at[1, sl], tmp1, st1)

    def red_step(j, tmp, st):
        pltpu.make_async_copy(shared.at[j, sl], tmp, st).wait()

        def add_vec(i, _):
            v = pl.ds(i * 16, 16)
            acc[v] = acc[v] + tmp[v]
            return ()
        lax.fori_loop(0, SLICE // 16, add_vec, ())

        @pl.when(j + 2 < NS)
        def _():
            pltpu.async_copy(shared.at[j + 2, sl], tmp, st)

    def red_body(j, _):
        red_step(2 * j, tmp0, st0)
        red_step(2 * j + 1, tmp1, st1)
        return ()
    lax.fori_loop(0, NS // 2, red_body, ())

    pltpu.sync_copy(acc, out_hbm.at[c, sl])


@functools.partial(
    pl.kernel,
    out_type=jax.ShapeDtypeStruct((NC, NP, OUT_CH), jnp.float32),
    mesh=_mesh,
    scratch_types=[
        pltpu.VMEM((NCHUNK, CH), jnp.int32),
        pltpu.VMEM((NCHUNK, CH), jnp.int32),
        pltpu.VMEM((CH, OUT_CH), jnp.float32),
        pltpu.VMEM((CH, OUT_CH), jnp.float32),
        pltpu.VMEM((CH, OUT_CH), jnp.float32),
        pltpu.VMEM((CH, OUT_CH), jnp.float32),
        pltpu.VMEM((SLICE, OUT_CH), jnp.float32),
        pltpu.VMEM_SHARED((NP, OUT_CH), jnp.float32),
        pltpu.SemaphoreType.DMA,
        pltpu.SemaphoreType.DMA,
        pltpu.SemaphoreType.DMA,
        pltpu.SemaphoreType.DMA,
        pltpu.SemaphoreType.DMA,
        pltpu.SemaphoreType.DMA,
        pltpu.SemaphoreType.DMA,
        pltpu.SemaphoreType.DMA,
        pltpu.SemaphoreType.DMA,
        pltpu.SemaphoreType.DMA,
    ],
    compiler_params=pltpu.CompilerParams(
        needs_layout_passes=False, use_tc_tiling_on_sc=False),
)
def _aggregate_kernel(g_hbm, e3_hbm, out_hbm,
                      srcall, dstall, rows0, rows1, rows2, rows3, tmp, acc,
                      si0, si1, sg0, sg1, sg2, sg3, ss0, ss1, ss2, ss3):
    c = lax.axis_index("c")
    s = lax.axis_index("s")
    wid = s * NC + c
    zeros16 = jnp.zeros((16,), jnp.float32)

    rows = (rows0, rows1, rows2, rows3)
    sg = (sg0, sg1, sg2, sg3)
    ss = (ss0, ss1, ss2, ss3)

    pltpu.async_copy(e3_hbm.at[0, wid], srcall, si0)
    pltpu.async_copy(e3_hbm.at[1, wid], dstall, si1)

    def zero_row(i, _):
        def zero_col(j, _):
            tmp[i, pl.ds(j * 16, 16)] = zeros16
            return ()
        lax.fori_loop(0, OUT_CH // 16, zero_col, ())
        return ()
    lax.fori_loop(0, SLICE, zero_row, ())
    pltpu.sync_copy(tmp, acc.at[pl.ds(s * SLICE, SLICE)])
    plsc.subcore_barrier()

    pltpu.make_async_copy(e3_hbm.at[0, wid], srcall, si0).wait()
    pltpu.make_async_copy(e3_hbm.at[1, wid], dstall, si1).wait()
    pltpu.async_copy(g_hbm.at[srcall.at[0]], rows0, sg0)
    pltpu.async_copy(g_hbm.at[srcall.at[1]], rows1, sg1)

    def wait_gather(a, b):
        pltpu.make_async_copy(g_hbm.at[srcall.at[a]], rows[b], sg[b]).wait()

    def issue_scatter(a, b):
        pltpu.async_copy(rows[b], acc.at[dstall.at[a]], ss[b], add=True)

    def wait_scatter(a, b):
        pltpu.make_async_copy(rows[b], acc.at[dstall.at[a]], ss[b]).wait()

    def issue_gather(a, b):
        @pl.when(a < NCHUNK)
        def _():
            pltpu.async_copy(g_hbm.at[srcall.at[a]], rows[b], sg[b])

    wait_gather(0, 0)
    issue_scatter(0, 0)
    issue_gather(2, 2)
    wait_gather(1, 1)
    issue_scatter(1, 1)
    issue_gather(3, 3)

    def step(i, b, bprev):
        wait_gather(i, b)
        issue_scatter(i, b)
        wait_scatter(i - 2, bprev)
        issue_gather(i + 2, bprev)

    def body(j, _):
        i = 4 * j + 2
        step(i, 2, 0)
        step(i + 1, 3, 1)
        step(i + 2, 0, 2)
        step(i + 3, 1, 3)
        return ()
    lax.fori_loop(0, (NCHUNK - 5) // 4, body, ())

    wait_gather(NCHUNK - 3, 2)
    issue_scatter(NCHUNK - 3, 2)
    wait_scatter(NCHUNK - 5, 0)
    issue_gather(NCHUNK - 1, 0)
    wait_gather(NCHUNK - 2, 3)
    issue_scatter(NCHUNK - 2, 3)
    wait_scatter(NCHUNK - 4, 1)
    wait_gather(NCHUNK - 1, 0)
    issue_scatter(NCHUNK - 1, 0)
    wait_scatter(NCHUNK - 3, 2)
    wait_scatter(NCHUNK - 2, 3)
    wait_scatter(NCHUNK - 1, 0)

    plsc.subcore_barrier()
    pltpu.sync_copy(acc.at[pl.ds(s * SLICE, SLICE)], tmp)
    pltpu.sync_copy(tmp, out_hbm.at[c, pl.ds(s * SLICE, SLICE)])


def _encode_body(x_ref, w_ref, dis_ref, g_ref):
    h = jnp.dot(x_ref[...], w_ref[...], preferred_element_type=jnp.float32)
    g_ref[...] = h * dis_ref[...].reshape(N_NODES, 1)


def _encode(x, w, dis):
    return pl.pallas_call(
        _encode_body,
        out_shape=jax.ShapeDtypeStruct((N_NODES, OUT_CH), jnp.float32),
    )(x, w, dis)


def _final_body(p_ref, g_ref, dis_ref, b_ref, o_ref):
    d = dis_ref[...].reshape(N_NODES, 1)
    o_ref[...] = d * (p_ref[0, :N_NODES] + p_ref[1, :N_NODES] + g_ref[...]) + b_ref[...]


def _finalize(partial, g, dis, b2d):
    return pl.pallas_call(
        _final_body,
        out_shape=jax.ShapeDtypeStruct((N_NODES, OUT_CH), jnp.float32),
    )(partial, g, dis, b2d)


@jax.jit
def kernel(x, edge_index, W, b):
    e3 = edge_index.astype(jnp.int32).reshape(2, NW, NCHUNK, CH)

    hist2 = _degree_kernel(e3)
    deg = hist2[0, :N_NODES] + hist2[1, :N_NODES] + 1.0
    dis = lax.rsqrt(deg)

    g = _encode(x, W, dis)
    partial = _aggregate_kernel(g, e3)
    return _finalize(partial, g, dis, b.reshape(1, OUT_CH))


# speedup vs baseline: 1.1206x; 1.1206x over previous
import functools

import jax
import jax.numpy as jnp
from jax import lax
from jax.experimental import pallas as pl
from jax.experimental.pallas import tpu as pltpu
from jax.experimental.pallas import tpu_sc as plsc

N_NODES = 10000
N_EDGES = 320000
IN_CH = 128
OUT_CH = 64

NC = 2
NS = 16
NW = NC * NS
NP = 10240
EPW = N_EDGES // NW
CH = 80
NCHUNK = EPW // CH
SLICE = NP // NS
CHA = 400
CW = 128

_mesh = plsc.VectorSubcoreMesh(
    core_axis_name="c", subcore_axis_name="s", num_cores=NC, num_subcores=NS)


@functools.partial(
    pl.kernel,
    out_type=jax.ShapeDtypeStruct((NC, NP), jnp.float32),
    mesh=_mesh,
    scratch_types=[
        pltpu.VMEM((NCHUNK, CH), jnp.int32),
        pltpu.VMEM((NP,), jnp.float32),
        pltpu.VMEM_SHARED((NS, NP), jnp.float32),
        pltpu.VMEM((SLICE,), jnp.float32),
        pltpu.VMEM((SLICE,), jnp.float32),
        pltpu.VMEM((SLICE,), jnp.float32),
        pltpu.SemaphoreType.DMA,
        pltpu.SemaphoreType.DMA,
        pltpu.SemaphoreType.DMA,
    ],
    compiler_params=pltpu.CompilerParams(
        needs_layout_passes=False, use_tc_tiling_on_sc=False),
)
def _degree_kernel(e3_hbm, out_hbm, dstall, hist, shared, acc, tmp0, tmp1,
                   si, st0, st1):
    c = lax.axis_index("c")
    s = lax.axis_index("s")
    wid = s * NC + c
    zeros16 = jnp.zeros((16,), jnp.float32)
    ones16 = jnp.ones((16,), jnp.float32)

    pltpu.async_copy(e3_hbm.at[1, wid], dstall, si)

    def zero_hist(i, _):
        hist[pl.ds(i * 16, 16)] = zeros16
        return ()
    lax.fori_loop(0, NP // 16, zero_hist, ())
    pltpu.make_async_copy(e3_hbm.at[1, wid], dstall, si).wait()

    def outer(j, _):
        def inner(k, _):
            idx = dstall[j, pl.ds(k * 16, 16)]
            plsc.addupdate_scatter(hist, [idx], ones16)
            return ()
        lax.fori_loop(0, CH // 16, inner, ())
        return ()
    lax.fori_loop(0, NCHUNK, outer, ())

    pltpu.sync_copy(hist, shared.at[s])
    plsc.subcore_barrier()

    def zero_acc(i, _):
        acc[pl.ds(i * 16, 16)] = zeros16
        return ()
    lax.fori_loop(0, SLICE // 16, zero_acc, ())

    sl = pl.ds(s * SLICE, SLICE)
    pltpu.async_copy(shared.at[0, sl], tmp0, st0)
    pltpu.async_copy(shared.at[1, sl], tmp1, st1)

    def red_step(j, tmp, st):
        pltpu.make_async_copy(shared.at[j, sl], tmp, st).wait()

        def add_vec(i, _):
            v = pl.ds(i * 16, 16)
            acc[v] = acc[v] + tmp[v]
            return ()
        lax.fori_loop(0, SLICE // 16, add_vec, ())

        @pl.when(j + 2 < NS)
        def _():
            pltpu.async_copy(shared.at[j + 2, sl], tmp, st)

    def red_body(j, _):
        red_step(2 * j, tmp0, st0)
        red_step(2 * j + 1, tmp1, st1)
        return ()
    lax.fori_loop(0, NS // 2, red_body, ())

    pltpu.sync_copy(acc, out_hbm.at[c, sl])


@functools.partial(
    pl.kernel,
    out_type=jax.ShapeDtypeStruct((NC, NP, OUT_CH), jnp.float32),
    mesh=_mesh,
    scratch_types=[
        pltpu.VMEM((NCHUNK, CH), jnp.int32),
        pltpu.VMEM((NCHUNK, CH), jnp.int32),
        pltpu.VMEM((CH, OUT_CH), jnp.float32),
        pltpu.VMEM((CH, OUT_CH), jnp.float32),
        pltpu.VMEM((CH, OUT_CH), jnp.float32),
        pltpu.VMEM((SLICE, OUT_CH), jnp.float32),
        pltpu.VMEM_SHARED((NP, OUT_CH), jnp.float32),
        pltpu.SemaphoreType.DMA,
        pltpu.SemaphoreType.DMA,
        pltpu.SemaphoreType.DMA,
        pltpu.SemaphoreType.DMA,
        pltpu.SemaphoreType.DMA,
        pltpu.SemaphoreType.DMA,
        pltpu.SemaphoreType.DMA,
        pltpu.SemaphoreType.DMA,
    ],
    compiler_params=pltpu.CompilerParams(
        needs_layout_passes=False, use_tc_tiling_on_sc=False),
)
def _aggregate_kernel(g_hbm, e3_hbm, out_hbm,
                      srcall, dstall, rows0, rows1, rows2, tmp, acc,
                      si0, si1, sg0, sg1, sg2, ss0, ss1, ss2):
    c = lax.axis_index("c")
    s = lax.axis_index("s")
    wid = s * NC + c
    zeros16 = jnp.zeros((16,), jnp.float32)

    rows = (rows0, rows1, rows2)
    sg = (sg0, sg1, sg2)
    ss = (ss0, ss1, ss2)

    pltpu.async_copy(e3_hbm.at[0, wid], srcall, si0)
    pltpu.async_copy(e3_hbm.at[1, wid], dstall, si1)

    def zero_row(i, _):
        def zero_col(j, _):
            tmp[i, pl.ds(j * 16, 16)] = zeros16
            return ()
        lax.fori_loop(0, OUT_CH // 16, zero_col, ())
        return ()
    lax.fori_loop(0, SLICE, zero_row, ())
    pltpu.sync_copy(tmp, acc.at[pl.ds(s * SLICE, SLICE)])
    plsc.subcore_barrier()

    pltpu.make_async_copy(e3_hbm.at[0, wid], srcall, si0).wait()
    pltpu.make_async_copy(e3_hbm.at[1, wid], dstall, si1).wait()
    pltpu.async_copy(g_hbm.at[srcall.at[0]], rows0, sg0)
    pltpu.async_copy(g_hbm.at[srcall.at[1]], rows1, sg1)

    def wait_gather(a, b):
        pltpu.make_async_copy(g_hbm.at[srcall.at[a]], rows[b], sg[b]).wait()

    def issue_scatter(a, b):
        pltpu.async_copy(rows[b], acc.at[dstall.at[a]], ss[b], add=True)

    def wait_scatter(a, b):
        pltpu.make_async_copy(rows[b], acc.at[dstall.at[a]], ss[b]).wait()

    def issue_gather(a, b):
        @pl.when(a < NCHUNK)
        def _():
            pltpu.async_copy(g_hbm.at[srcall.at[a]], rows[b], sg[b])

    wait_gather(0, 0)
    issue_scatter(0, 0)
    issue_gather(2, 2)

    def step(i, b, bprev):
        wait_gather(i, b)
        issue_scatter(i, b)
        wait_scatter(i - 1, bprev)
        issue_gather(i + 2, bprev)

    def body(j, _):
        i = 3 * j + 1
        step(i, 1, 0)
        step(i + 1, 2, 1)
        step(i + 2, 0, 2)
        return ()
    lax.fori_loop(0, (NCHUNK - 2) // 3, body, ())

    wait_gather(NCHUNK - 1, 1)
    issue_scatter(NCHUNK - 1, 1)
    wait_scatter(NCHUNK - 2, 0)
    wait_scatter(NCHUNK - 1, 1)

    plsc.subcore_barrier()
    pltpu.sync_copy(acc.at[pl.ds(s * SLICE, SLICE)], tmp)
    pltpu.sync_copy(tmp, out_hbm.at[c, pl.ds(s * SLICE, SLICE)])


def _encode_body(x_ref, w_ref, dis_ref, g_ref):
    h = jnp.dot(x_ref[...], w_ref[...], preferred_element_type=jnp.float32)
    g_ref[...] = h * dis_ref[...].reshape(N_NODES, 1)


def _encode(x, w, dis):
    return pl.pallas_call(
        _encode_body,
        out_shape=jax.ShapeDtypeStruct((N_NODES, OUT_CH), jnp.float32),
    )(x, w, dis)


_NLV = N_NODES * OUT_CH // 128


def _final_body(p_ref, g_ref, d0_ref, d1_ref, b_ref, o_ref):
    q = p_ref[0, :_NLV] + p_ref[1, :_NLV] + g_ref[...]
    d0 = jnp.broadcast_to(d0_ref[...].reshape(_NLV, 1), (_NLV, OUT_CH))
    d1 = jnp.broadcast_to(d1_ref[...].reshape(_NLV, 1), (_NLV, OUT_CH))
    d = jnp.concatenate([d0, d1], axis=1)
    o_ref[...] = d * q + b_ref[...]


def _finalize(partial128, g128, dis0, dis1, b128):
    return pl.pallas_call(
        _final_body,
        out_shape=jax.ShapeDtypeStruct((_NLV, 128), jnp.float32),
    )(partial128, g128, dis0, dis1, b128)


@jax.jit
def kernel(x, edge_index, W, b):
    e3 = edge_index.astype(jnp.int32).reshape(2, NW, NCHUNK, CH)

    hist2 = _degree_kernel(e3)
    deg = hist2[0, :N_NODES] + hist2[1, :N_NODES] + 1.0
    dis = lax.rsqrt(deg)

    g = _encode(x, W, dis)
    partial = _aggregate_kernel(g, e3)
    partial128 = partial.reshape(NC, NP * OUT_CH // 128, 128)
    g128 = g.reshape(_NLV, 128)
    b128 = jnp.concatenate([b, b]).reshape(1, 128)
    o128 = _finalize(partial128, g128, dis[0::2], dis[1::2], b128)
    return o128.reshape(N_NODES, OUT_CH)


# speedup vs baseline: 1.1210x; 1.0003x over previous
import functools

import jax
import jax.numpy as jnp
from jax import lax
from jax.experimental import pallas as pl
from jax.experimental.pallas import tpu as pltpu
from jax.experimental.pallas import tpu_sc as plsc

N_NODES = 10000
N_EDGES = 320000
IN_CH = 128
OUT_CH = 64

NC = 2
NS = 16
NW = NC * NS
NP = 10240
EPW = N_EDGES // NW
CH = 80
NCHUNK = EPW // CH
SLICE = NP // NS

_mesh = plsc.VectorSubcoreMesh(
    core_axis_name="c", subcore_axis_name="s", num_cores=NC, num_subcores=NS)


@functools.partial(
    pl.kernel,
    out_type=jax.ShapeDtypeStruct((NC, NP), jnp.float32),
    mesh=_mesh,
    scratch_types=[
        pltpu.VMEM((NCHUNK, CH), jnp.int32),
        pltpu.VMEM((NP,), jnp.float32),
        pltpu.VMEM_SHARED((NS, NP), jnp.float32),
        pltpu.VMEM((SLICE,), jnp.float32),
        pltpu.VMEM((SLICE,), jnp.float32),
        pltpu.VMEM((SLICE,), jnp.float32),
        pltpu.SemaphoreType.DMA,
        pltpu.SemaphoreType.DMA,
        pltpu.SemaphoreType.DMA,
    ],
    compiler_params=pltpu.CompilerParams(
        needs_layout_passes=False, use_tc_tiling_on_sc=False),
)
def _degree_kernel(e3_hbm, out_hbm, dstall, hist, shared, acc, tmp0, tmp1,
                   si, st0, st1):
    c = lax.axis_index("c")
    s = lax.axis_index("s")
    wid = s * NC + c
    zeros16 = jnp.zeros((16,), jnp.float32)
    ones16 = jnp.ones((16,), jnp.float32)

    pltpu.async_copy(e3_hbm.at[1, wid], dstall, si)

    def zero_hist(i, _):
        hist[pl.ds(i * 16, 16)] = zeros16
        return ()
    lax.fori_loop(0, NP // 16, zero_hist, ())
    pltpu.make_async_copy(e3_hbm.at[1, wid], dstall, si).wait()

    def outer(j, _):
        def inner(k, _):
            idx = dstall[j, pl.ds(k * 16, 16)]
            plsc.addupdate_scatter(hist, [idx], ones16)
            return ()
        lax.fori_loop(0, CH // 16, inner, ())
        return ()
    lax.fori_loop(0, NCHUNK, outer, ())

    pltpu.sync_copy(hist, shared.at[s])
    plsc.subcore_barrier()

    def zero_acc(i, _):
        acc[pl.ds(i * 16, 16)] = zeros16
        return ()
    lax.fori_loop(0, SLICE // 16, zero_acc, ())

    sl = pl.ds(s * SLICE, SLICE)
    pltpu.async_copy(shared.at[0, sl], tmp0, st0)
    pltpu.async_copy(shared.at[1, sl], tmp1, st1)

    def red_step(j, tmp, st):
        pltpu.make_async_copy(shared.at[j, sl], tmp, st).wait()

        def add_vec(i, _):
            v = pl.ds(i * 16, 16)
            acc[v] = acc[v] + tmp[v]
            return ()
        lax.fori_loop(0, SLICE // 16, add_vec, ())

        @pl.when(j + 2 < NS)
        def _():
            pltpu.async_copy(shared.at[j + 2, sl], tmp, st)

    def red_body(j, _):
        red_step(2 * j, tmp0, st0)
        red_step(2 * j + 1, tmp1, st1)
        return ()
    lax.fori_loop(0, NS // 2, red_body, ())

    pltpu.sync_copy(acc, out_hbm.at[c, sl])


@functools.partial(
    pl.kernel,
    out_type=jax.ShapeDtypeStruct((NC, NP, OUT_CH), jnp.float32),
    mesh=_mesh,
    scratch_types=[
        pltpu.VMEM((NCHUNK, CH), jnp.int32),
        pltpu.VMEM((NCHUNK, CH), jnp.int32),
        pltpu.VMEM((CH, OUT_CH), jnp.float32),
        pltpu.VMEM((CH, OUT_CH), jnp.float32),
        pltpu.VMEM((CH, OUT_CH), jnp.float32),
        pltpu.VMEM((SLICE, OUT_CH), jnp.float32),
        pltpu.VMEM_SHARED((NP, OUT_CH), jnp.float32),
        pltpu.SemaphoreType.DMA,
        pltpu.SemaphoreType.DMA,
        pltpu.SemaphoreType.DMA,
        pltpu.SemaphoreType.DMA,
        pltpu.SemaphoreType.DMA,
        pltpu.SemaphoreType.DMA,
        pltpu.SemaphoreType.DMA,
        pltpu.SemaphoreType.DMA,
    ],
    compiler_params=pltpu.CompilerParams(
        needs_layout_passes=False, use_tc_tiling_on_sc=False),
)
def _aggregate_kernel(g_hbm, e3_hbm, out_hbm,
                      srcall, dstall, rows0, rows1, rows2, tmp, acc,
                      si0, si1, sg0, sg1, sg2, ss0, ss1, ss2):
    c = lax.axis_index("c")
    s = lax.axis_index("s")
    wid = s * NC + c
    zeros16 = jnp.zeros((16,), jnp.float32)

    rows = (rows0, rows1, rows2)
    sg = (sg0, sg1, sg2)
    ss = (ss0, ss1, ss2)

    pltpu.async_copy(e3_hbm.at[0, wid], srcall, si0)
    pltpu.async_copy(e3_hbm.at[1, wid], dstall, si1)

    def zero_row(i, _):
        def zero_col(j, _):
            tmp[i, pl.ds(j * 16, 16)] = zeros16
            return ()
        lax.fori_loop(0, OUT_CH // 16, zero_col, ())
        return ()
    lax.fori_loop(0, SLICE, zero_row, ())
    pltpu.sync_copy(tmp, acc.at[pl.ds(s * SLICE, SLICE)])
    plsc.subcore_barrier()

    pltpu.make_async_copy(e3_hbm.at[0, wid], srcall, si0).wait()
    pltpu.make_async_copy(e3_hbm.at[1, wid], dstall, si1).wait()
    pltpu.async_copy(g_hbm.at[srcall.at[0]], rows0, sg0)
    pltpu.async_copy(g_hbm.at[srcall.at[1]], rows1, sg1)

    def wait_gather(a, b):
        pltpu.make_async_copy(g_hbm.at[srcall.at[a]], rows[b], sg[b]).wait()

    def issue_scatter(a, b):
        pltpu.async_copy(rows[b], acc.at[dstall.at[a]], ss[b], add=True)

    def wait_scatter(a, b):
        pltpu.make_async_copy(rows[b], acc.at[dstall.at[a]], ss[b]).wait()

    def issue_gather(a, b):
        @pl.when(a < NCHUNK)
        def _():
            pltpu.async_copy(g_hbm.at[srcall.at[a]], rows[b], sg[b])

    wait_gather(0, 0)
    issue_scatter(0, 0)
    issue_gather(2, 2)

    def step(i, b, bprev):
        wait_gather(i, b)
        issue_scatter(i, b)
        wait_scatter(i - 1, bprev)
        issue_gather(i + 2, bprev)

    def body(j, _):
        i = 3 * j + 1
        step(i, 1, 0)
        step(i + 1, 2, 1)
        step(i + 2, 0, 2)
        return ()
    lax.fori_loop(0, (NCHUNK - 2) // 3, body, ())

    wait_gather(NCHUNK - 1, 1)
    issue_scatter(NCHUNK - 1, 1)
    wait_scatter(NCHUNK - 2, 0)
    wait_scatter(NCHUNK - 1, 1)

    plsc.subcore_barrier()
    pltpu.sync_copy(acc.at[pl.ds(s * SLICE, SLICE)], tmp)
    pltpu.sync_copy(tmp, out_hbm.at[c, pl.ds(s * SLICE, SLICE)])


def _encode_body(x_ref, w_ref, dis_ref, g_ref):
    h = jnp.dot(x_ref[...], w_ref[...], preferred_element_type=jnp.float32)
    g_ref[...] = h * dis_ref[...].reshape(N_NODES, 1)


def _encode(x, w, dis):
    return pl.pallas_call(
        _encode_body,
        out_shape=jax.ShapeDtypeStruct((N_NODES, OUT_CH), jnp.float32),
    )(x, w, dis)


_NLV = N_NODES * OUT_CH // 128


def _final_body(p_ref, g_ref, d0_ref, d1_ref, b_ref, o_ref):
    q = p_ref[0, :_NLV] + p_ref[1, :_NLV] + g_ref[...]
    d0 = jnp.broadcast_to(d0_ref[...].reshape(_NLV, 1), (_NLV, OUT_CH))
    d1 = jnp.broadcast_to(d1_ref[...].reshape(_NLV, 1), (_NLV, OUT_CH))
    d = jnp.concatenate([d0, d1], axis=1)
    o_ref[...] = d * q + b_ref[...]


def _finalize(partial128, g128, dis0, dis1, b128):
    return pl.pallas_call(
        _final_body,
        out_shape=jax.ShapeDtypeStruct((_NLV, 128), jnp.float32),
    )(partial128, g128, dis0, dis1, b128)


@jax.jit
def kernel(x, edge_index, W, b):
    e3 = edge_index.astype(jnp.int32).reshape(2, NW, NCHUNK, CH)

    hist2 = _degree_kernel(e3)
    deg = hist2[0, :N_NODES] + hist2[1, :N_NODES] + 1.0
    dis = lax.rsqrt(deg)

    g = _encode(x, W, dis)
    partial = _aggregate_kernel(g, e3)
    partial128 = partial.reshape(NC, NP * OUT_CH // 128, 128)
    g128 = g.reshape(_NLV, 128)
    b128 = jnp.concatenate([b, b]).reshape(1, 128)
    o128 = _finalize(partial128, g128, dis[0::2], dis[1::2], b128)
    return o128.reshape(N_NODES, OUT_CH)
